# baseline (speedup 1.0000x reference)
import functools

import jax
import jax.numpy as jnp
from jax import lax
from jax.experimental import pallas as pl
from jax.experimental.pallas import tpu as pltpu
from jax.experimental.pallas import tpu_sc as plsc

_B = 16384
_PAD = 50
_D = 64
_V = 60
_BB = 2048
_NB = _B // _BB


def _tc_body(idx_ref, tab_ref, out_ref):
    idx = idx_ref[0, 0, :]
    oh = (lax.broadcasted_iota(jnp.int32, (_D, _BB), 0)
          == idx[None, :]).astype(jnp.float32)
    out_ref[0] = jnp.dot(tab_ref[...], oh,
                         precision=lax.Precision.HIGHEST,
                         preferred_element_type=jnp.float32)


_tc_emb = pl.pallas_call(
    _tc_body,
    grid=(_PAD, _NB),
    in_specs=[
        pl.BlockSpec((1, 1, _BB), lambda p, ib: (p, 0, ib)),
        pl.BlockSpec((_D, _D), lambda p, ib: (0, 0)),
    ],
    out_specs=pl.BlockSpec((1, _D, _BB), lambda p, ib: (p, 0, ib)),
    out_shape=jax.ShapeDtypeStruct((_PAD, _D, _B), jnp.float32),
)


@jax.jit
def kernel(indices, emb_weight):
    idx_t = indices.T.reshape(_PAD, 1, _B)
    tab_t = jnp.pad(emb_weight, ((0, _D - _V), (0, 0))).T
    out_t = _tc_emb(idx_t, tab_t)
    return out_t.transpose(2, 0, 1)


# speedup vs baseline: 1.9155x; 1.9155x over previous
import functools

import jax
import jax.numpy as jnp
from jax import lax
from jax.experimental import pallas as pl
from jax.experimental.pallas import tpu as pltpu
from jax.experimental.pallas import tpu_sc as plsc

_B = 16384
_PAD = 50
_D = 64
_V = 60
_BB = 4096
_NB = _B // _BB


def _tc_body(idx_ref, tab_ref, out_ref):
    idx = idx_ref[0, 0, :]
    oh = (lax.broadcasted_iota(jnp.int32, (_D, _BB), 0)
          == idx[None, :]).astype(jnp.bfloat16)
    out_ref[0] = jnp.dot(tab_ref[...], oh,
                         preferred_element_type=jnp.float32)


_tc_emb = pl.pallas_call(
    _tc_body,
    grid=(_PAD, _NB),
    in_specs=[
        pl.BlockSpec((1, 1, _BB), lambda p, ib: (p, 0, ib)),
        pl.BlockSpec((_D, _D), lambda p, ib: (0, 0)),
    ],
    out_specs=pl.BlockSpec((1, _D, _BB), lambda p, ib: (p, 0, ib)),
    out_shape=jax.ShapeDtypeStruct((_PAD, _D, _B), jnp.float32),
)


@jax.jit
def kernel(indices, emb_weight):
    idx_t = indices.T.reshape(_PAD, 1, _B)
    tab_t = jnp.pad(emb_weight, ((0, _D - _V), (0, 0))).T.astype(jnp.bfloat16)
    out_t = _tc_emb(idx_t, tab_t)
    return out_t.transpose(2, 0, 1)


# speedup vs baseline: 2.8264x; 1.4755x over previous
import functools

import jax
import jax.numpy as jnp
from jax import lax
from jax.experimental import pallas as pl
from jax.experimental.pallas import tpu as pltpu
from jax.experimental.pallas import tpu_sc as plsc

_B = 16384
_PAD = 50
_D = 64
_V = 60
_BB = 8192
_NB = _B // _BB


def _tc_body(idx_ref, tab_ref, out_ref):
    idx = idx_ref[0, 0, :]
    oh = (lax.broadcasted_iota(jnp.int32, (_D, _BB), 0)
          == idx[None, :]).astype(jnp.bfloat16)
    out_ref[0] = jnp.dot(tab_ref[...], oh,
                         preferred_element_type=jnp.float32)


_tc_emb = pl.pallas_call(
    _tc_body,
    grid=(_PAD, _NB),
    in_specs=[
        pl.BlockSpec((1, 1, _BB), lambda p, ib: (p, 0, ib)),
        pl.BlockSpec((_D, _D), lambda p, ib: (0, 0)),
    ],
    out_specs=pl.BlockSpec((1, _D, _BB), lambda p, ib: (p, 0, ib)),
    out_shape=jax.ShapeDtypeStruct((_PAD, _D, _B), jnp.float32),
)


@jax.jit
def kernel(indices, emb_weight):
    idx_t = indices.T.reshape(_PAD, 1, _B)
    tab_t = jnp.pad(emb_weight, ((0, _D - _V), (0, 0))).T.astype(jnp.bfloat16)
    out_t = _tc_emb(idx_t, tab_t)
    return out_t.transpose(2, 0, 1)


# speedup vs baseline: 3.7975x; 1.3436x over previous
import functools

import jax
import jax.numpy as jnp
from jax import lax
from jax.experimental import pallas as pl
from jax.experimental.pallas import tpu as pltpu
from jax.experimental.pallas import tpu_sc as plsc

_B = 16384
_PAD = 50
_D = 64
_V = 60
_BB = 16384
_NB = _B // _BB


def _tc_body(idx_ref, tab_ref, out_ref):
    idx = idx_ref[0, 0, :]
    oh = (lax.broadcasted_iota(jnp.int32, (_D, _BB), 0)
          == idx[None, :]).astype(jnp.bfloat16)
    out_ref[0] = jnp.dot(tab_ref[...], oh,
                         preferred_element_type=jnp.float32)


_tc_emb = pl.pallas_call(
    _tc_body,
    grid=(_PAD, _NB),
    in_specs=[
        pl.BlockSpec((1, 1, _BB), lambda p, ib: (p, 0, ib)),
        pl.BlockSpec((_D, _D), lambda p, ib: (0, 0)),
    ],
    out_specs=pl.BlockSpec((1, _D, _BB), lambda p, ib: (p, 0, ib)),
    out_shape=jax.ShapeDtypeStruct((_PAD, _D, _B), jnp.float32),
)


@jax.jit
def kernel(indices, emb_weight):
    idx_t = indices.T.reshape(_PAD, 1, _B)
    tab_t = jnp.pad(emb_weight, ((0, _D - _V), (0, 0))).T.astype(jnp.bfloat16)
    out_t = _tc_emb(idx_t, tab_t)
    return out_t.transpose(2, 0, 1)


# speedup vs baseline: 4.3817x; 1.1538x over previous
import functools

import jax
import jax.numpy as jnp
from jax import lax
from jax.experimental import pallas as pl
from jax.experimental.pallas import tpu as pltpu
from jax.experimental.pallas import tpu_sc as plsc

_B = 16384
_PAD = 50
_D = 64
_V = 60
_BB = 4096
_NB = _B // _BB
_PB = 8
_NP = (_PAD + _PB - 1) // _PB


def _tc_body(idx_ref, tab_ref, out_ref):
    for q in range(_PB):
        idx = idx_ref[q, :]
        oh = (lax.broadcasted_iota(jnp.int32, (_D, _BB), 0)
              == idx[None, :]).astype(jnp.bfloat16)
        out_ref[q] = jnp.dot(tab_ref[...], oh,
                             preferred_element_type=jnp.float32)


_tc_emb = pl.pallas_call(
    _tc_body,
    grid=(_NP, _NB),
    in_specs=[
        pl.BlockSpec((_PB, _BB), lambda pb, ib: (pb, ib)),
        pl.BlockSpec((_D, _D), lambda pb, ib: (0, 0)),
    ],
    out_specs=pl.BlockSpec((_PB, _D, _BB), lambda pb, ib: (pb, 0, ib)),
    out_shape=jax.ShapeDtypeStruct((_PAD, _D, _B), jnp.float32),
)


@jax.jit
def kernel(indices, emb_weight):
    idx_t = indices.T
    tab_t = jnp.pad(emb_weight, ((0, _D - _V), (0, 0))).T.astype(jnp.bfloat16)
    out_t = _tc_emb(idx_t, tab_t)
    return out_t.transpose(2, 0, 1)
